# baseline (speedup 1.0000x reference)
import functools

import jax
import jax.numpy as jnp
from jax import lax
from jax.experimental import pallas as pl
from jax.experimental.pallas import tpu as pltpu
from jax.experimental.pallas import tpu_sc as plsc

_NC = 2
_NS = 16
_NW = _NC * _NS
_K = 80


def _sc_mesh():
    return plsc.VectorSubcoreMesh(
        core_axis_name="c", subcore_axis_name="s", num_cores=_NC, num_subcores=_NS
    )


def _row_split(np_):
    rpt = (np_ // _NS) // 8 * 8
    tail = np_ - _NS * rpt
    return rpt, tail


@functools.cache
def _make_hist(np_, epw):
    kh = 128
    iters = epw // kh
    assert iters % 2 == 0
    rpt, tail = _row_split(np_)

    @functools.partial(
        pl.kernel,
        out_type=jax.ShapeDtypeStruct((_NC, np_, 128), jnp.float32),
        mesh=_sc_mesh(),
        scratch_types=[
            pltpu.VMEM((kh,), jnp.int32),
            pltpu.VMEM((kh,), jnp.int32),
            pltpu.VMEM((kh, 128), jnp.float32),
            pltpu.VMEM_SHARED((np_, 128), jnp.float32),
            pltpu.SemaphoreType.DMA,
            pltpu.SemaphoreType.DMA,
        ],
    )
    def hist(dst_hbm, zeros_hbm, out_hbm, idxa_v, idxb_v, ones_v, acc_sh, la, lb):
        c = lax.axis_index("c")
        s = lax.axis_index("s")
        wid = s * _NC + c
        for r in range(kh):
            for q in range(8):
                ones_v[r, pl.ds(q * 16, 16)] = jnp.ones((16,), jnp.float32)
        pltpu.sync_copy(
            zeros_hbm.at[pl.ds(s * rpt, rpt)], acc_sh.at[pl.ds(s * rpt, rpt)]
        )
        if tail:
            @pl.when(s == _NS - 1)
            def _():
                pltpu.sync_copy(
                    zeros_hbm.at[pl.ds(_NS * rpt, tail)],
                    acc_sh.at[pl.ds(_NS * rpt, tail)],
                )
        plsc.subcore_barrier()

        def load(chunk, idx_v, sem):
            pltpu.async_copy(
                dst_hbm.at[pl.ds(wid * epw + chunk * kh, kh)], idx_v, sem
            )

        def load_wait(idx_v, sem):
            pltpu.make_async_copy(dst_hbm.at[pl.ds(0, kh)], idx_v, sem).wait()

        pltpu.sync_copy(dst_hbm.at[pl.ds(wid * epw, kh)], idxa_v)

        def body(j, carry):
            j2 = j * 2
            load(j2 + 1, idxb_v, lb)
            pltpu.sync_copy(ones_v, acc_sh.at[idxa_v], add=True)
            load_wait(idxb_v, lb)
            load(jnp.minimum(j2 + 2, iters - 1), idxa_v, la)
            pltpu.sync_copy(ones_v, acc_sh.at[idxb_v], add=True)
            load_wait(idxa_v, la)
            return carry

        lax.fori_loop(0, iters // 2, body, 0)
        plsc.subcore_barrier()
        pltpu.sync_copy(
            acc_sh.at[pl.ds(s * rpt, rpt)], out_hbm.at[c, pl.ds(s * rpt, rpt)]
        )
        if tail:
            @pl.when(s == _NS - 1)
            def _():
                pltpu.sync_copy(
                    acc_sh.at[pl.ds(_NS * rpt, tail)],
                    out_hbm.at[c, pl.ds(_NS * rpt, tail)],
                )

    return hist


@functools.cache
def _make_msg(np_, iters, d):
    rpt, tail = _row_split(np_)
    phases = []
    rem = iters
    while rem:
        t = min(40, rem)
        phases.append(t)
        rem -= t
    assert all(p >= 2 and p % 8 == 0 for p in phases)
    pmax = max(phases)

    @functools.partial(
        pl.kernel,
        out_type=jax.ShapeDtypeStruct((_NC, np_, d), jnp.float32),
        mesh=_sc_mesh(),
        scratch_types=[
            pltpu.VMEM((pmax, _K), jnp.int32),
            pltpu.VMEM((pmax, _K), jnp.int32),
            pltpu.VMEM((_K, d), jnp.float32),
            pltpu.VMEM((_K, d), jnp.float32),
            pltpu.VMEM_SHARED((np_, d), jnp.float32),
            pltpu.SemaphoreType.DMA,
            pltpu.SemaphoreType.DMA,
            pltpu.SemaphoreType.DMA,
            pltpu.SemaphoreType.DMA,
        ],
    )
    def msg(hs_hbm, src_hbm, dst_hbm, zeros_hbm, out_hbm,
            sidx_v, didx_v, rows0_v, rows1_v, acc_sh, g0, g1, s0, s1):
        c = lax.axis_index("c")
        s = lax.axis_index("s")
        wid = s * _NC + c
        pltpu.sync_copy(
            zeros_hbm.at[pl.ds(s * rpt, rpt)], acc_sh.at[pl.ds(s * rpt, rpt)]
        )
        if tail:
            @pl.when(s == _NS - 1)
            def _():
                pltpu.sync_copy(
                    zeros_hbm.at[pl.ds(_NS * rpt, tail)],
                    acc_sh.at[pl.ds(_NS * rpt, tail)],
                )
        plsc.subcore_barrier()

        def g_start(j, rows_v, sem):
            pltpu.async_copy(hs_hbm.at[sidx_v.at[j]], rows_v, sem)

        def g_wait(rows_v, sem):
            pltpu.make_async_copy(hs_hbm.at[sidx_v.at[0]], rows_v, sem).wait()

        def s_start(j, rows_v, sem):
            pltpu.async_copy(rows_v, acc_sh.at[didx_v.at[j]], sem, add=True)

        def s_wait(rows_v, sem):
            pltpu.make_async_copy(rows_v, acc_sh.at[didx_v.at[0]], sem).wait()

        base = 0
        for count in phases:
            pltpu.sync_copy(
                src_hbm.at[wid, pl.ds(base, count)], sidx_v.at[pl.ds(0, count)]
            )
            pltpu.sync_copy(
                dst_hbm.at[wid, pl.ds(base, count)], didx_v.at[pl.ds(0, count)]
            )
            g_start(0, rows0_v, g0)

            def body(jj, carry):
                j2 = jj * 2
                g_wait(rows0_v, g0)
                @pl.when(jj > 0)
                def _():
                    s_wait(rows1_v, s1)
                g_start(j2 + 1, rows1_v, g1)
                s_start(j2, rows0_v, s0)
                g_wait(rows1_v, g1)
                s_wait(rows0_v, s0)
                g_start(jnp.minimum(j2 + 2, count - 1), rows0_v, g0)
                s_start(j2 + 1, rows1_v, s1)
                return carry

            lax.fori_loop(0, count // 2, body, 0)
            g_wait(rows0_v, g0)
            s_wait(rows1_v, s1)
            base += count
        plsc.subcore_barrier()
        pltpu.sync_copy(
            acc_sh.at[pl.ds(s * rpt, rpt)], out_hbm.at[c, pl.ds(s * rpt, rpt)]
        )
        if tail:
            @pl.when(s == _NS - 1)
            def _():
                pltpu.sync_copy(
                    acc_sh.at[pl.ds(_NS * rpt, tail)],
                    out_hbm.at[c, pl.ds(_NS * rpt, tail)],
                )

    return msg


def _scale_matmul_body(x_ref, cnt_ref, w_ref, o_ref):
    cnt = cnt_ref[...]
    deg = cnt[0, :, 0:1] + cnt[1, :, 0:1] + 1.0
    dinv = lax.rsqrt(deg)
    o_ref[...] = jnp.dot(
        x_ref[...] * dinv, w_ref[...], preferred_element_type=jnp.float32
    )


def _scale_matmul(x, cnt, w):
    n, d_in = x.shape
    d_h = w.shape[1]
    bm = 1000
    grid = n // bm
    return pl.pallas_call(
        _scale_matmul_body,
        grid=(grid,),
        in_specs=[
            pl.BlockSpec((bm, d_in), lambda i: (i, 0)),
            pl.BlockSpec((_NC, bm, 128), lambda i: (0, i, 0)),
            pl.BlockSpec((d_in, d_h), lambda i: (0, 0)),
        ],
        out_specs=pl.BlockSpec((bm, d_h), lambda i: (i, 0)),
        out_shape=jax.ShapeDtypeStruct((n, d_h), jnp.float32),
    )(x, cnt, w)


def _make_finalize(n):
    def _finalize_body(s_ref, hs_ref, cnt_ref, b_ref, g_ref, be_ref, z_ref):
        cnt = cnt_ref[...]
        deg = cnt[0, :n, 0:1] + cnt[1, :n, 0:1] + 1.0
        dinv = lax.rsqrt(deg)
        ssum = s_ref[0, :n, :] + s_ref[1, :n, :]
        agg = dinv * (ssum + hs_ref[...]) + b_ref[...]
        inv_n = 1.0 / n
        mean = jnp.sum(agg, axis=0, keepdims=True) * inv_n
        cen = agg - mean
        var = jnp.sum(cen * cen, axis=0, keepdims=True) * inv_n
        z = cen * lax.rsqrt(var + 1e-5) * g_ref[...] + be_ref[...]
        z_ref[...] = jnp.maximum(z, 0.0)

    return _finalize_body


def _finalize(s, hs, cnt, b, g, be):
    n, d = hs.shape
    return pl.pallas_call(
        _make_finalize(n),
        out_shape=jax.ShapeDtypeStruct((n, d), jnp.float32),
    )(s, hs, cnt, b, g, be)


def _decoder_body(zi_ref, zj_ref, o_ref):
    o_ref[...] = lax.dot_general(
        zi_ref[...],
        zj_ref[...],
        (((1,), (1,)), ((), ())),
        preferred_element_type=jnp.float32,
    )


def _decoder(z):
    n, d = z.shape
    bm = 512
    grid = pl.cdiv(n, bm)
    return pl.pallas_call(
        _decoder_body,
        grid=(grid, grid),
        in_specs=[
            pl.BlockSpec((bm, d), lambda i, j: (i, 0)),
            pl.BlockSpec((bm, d), lambda i, j: (j, 0)),
        ],
        out_specs=pl.BlockSpec((bm, bm), lambda i, j: (i, j)),
        out_shape=jax.ShapeDtypeStruct((n, n), jnp.float32),
        compiler_params=pltpu.CompilerParams(
            dimension_semantics=("parallel", "parallel")
        ),
    )(z, z)


def kernel(x, edge_index, W, b, gamma, beta):
    n, _ = x.shape
    d_h = W.shape[1]
    e = edge_index.shape[1]
    dummy = 512
    np_ = (n + dummy + 7) // 8 * 8
    chunks = -(-e // (_NW * _K))
    iters = -(-chunks // 8) * 8
    e_pad = _NW * _K * iters
    src = jnp.concatenate(
        [edge_index[0], jnp.zeros((e_pad - e,), jnp.int32)]
    ).reshape(_NW, iters, _K)
    dst_flat = jnp.concatenate(
        [edge_index[1], n + jnp.arange(e_pad - e, dtype=jnp.int32) % dummy]
    )
    dst = dst_flat.reshape(_NW, iters, _K)
    zeros_d = jnp.zeros((np_, d_h), jnp.float32)
    cnt = _make_hist(np_, iters * _K)(dst_flat, zeros_d)
    hs = _scale_matmul(x, cnt, W)
    s = _make_msg(np_, iters, d_h)(hs, src, dst, zeros_d)
    z = _finalize(
        s, hs, cnt, b.reshape(1, d_h), gamma.reshape(1, d_h), beta.reshape(1, d_h)
    )
    return _decoder(z)


# speedup vs baseline: 1.6508x; 1.6508x over previous
import functools

import jax
import jax.numpy as jnp
from jax import lax
from jax.experimental import pallas as pl
from jax.experimental.pallas import tpu as pltpu
from jax.experimental.pallas import tpu_sc as plsc

_NC = 2
_NS = 16
_NW = _NC * _NS
_K = 80


def _sc_mesh():
    return plsc.VectorSubcoreMesh(
        core_axis_name="c", subcore_axis_name="s", num_cores=_NC, num_subcores=_NS
    )


def _row_split(np_):
    rpt = (np_ // _NS) // 8 * 8
    tail = np_ - _NS * rpt
    return rpt, tail


@functools.cache
def _make_hist(np_, epw):
    kh = 128
    iters = epw // kh
    assert iters % 2 == 0
    rpt, tail = _row_split(np_)

    @functools.partial(
        pl.kernel,
        out_type=jax.ShapeDtypeStruct((_NC, np_, 128), jnp.float32),
        mesh=_sc_mesh(),
        scratch_types=[
            pltpu.VMEM((kh,), jnp.int32),
            pltpu.VMEM((kh,), jnp.int32),
            pltpu.VMEM((kh, 128), jnp.float32),
            pltpu.VMEM_SHARED((np_, 128), jnp.float32),
            pltpu.SemaphoreType.DMA,
            pltpu.SemaphoreType.DMA,
        ],
    )
    def hist(dst_hbm, zeros_hbm, out_hbm, idxa_v, idxb_v, ones_v, acc_sh, la, lb):
        c = lax.axis_index("c")
        s = lax.axis_index("s")
        wid = s * _NC + c
        for r in range(kh):
            for q in range(8):
                ones_v[r, pl.ds(q * 16, 16)] = jnp.ones((16,), jnp.float32)
        pltpu.sync_copy(
            zeros_hbm.at[pl.ds(s * rpt, rpt)], acc_sh.at[pl.ds(s * rpt, rpt)]
        )
        if tail:
            @pl.when(s == _NS - 1)
            def _():
                pltpu.sync_copy(
                    zeros_hbm.at[pl.ds(_NS * rpt, tail)],
                    acc_sh.at[pl.ds(_NS * rpt, tail)],
                )
        plsc.subcore_barrier()

        def load(chunk, idx_v, sem):
            pltpu.async_copy(
                dst_hbm.at[pl.ds(wid * epw + chunk * kh, kh)], idx_v, sem
            )

        def load_wait(idx_v, sem):
            pltpu.make_async_copy(dst_hbm.at[pl.ds(0, kh)], idx_v, sem).wait()

        pltpu.sync_copy(dst_hbm.at[pl.ds(wid * epw, kh)], idxa_v)

        def body(j, carry):
            j2 = j * 2
            load(j2 + 1, idxb_v, lb)
            pltpu.sync_copy(ones_v, acc_sh.at[idxa_v], add=True)
            load_wait(idxb_v, lb)
            load(jnp.minimum(j2 + 2, iters - 1), idxa_v, la)
            pltpu.sync_copy(ones_v, acc_sh.at[idxb_v], add=True)
            load_wait(idxa_v, la)
            return carry

        lax.fori_loop(0, iters // 2, body, 0)
        plsc.subcore_barrier()
        pltpu.sync_copy(
            acc_sh.at[pl.ds(s * rpt, rpt)], out_hbm.at[c, pl.ds(s * rpt, rpt)]
        )
        if tail:
            @pl.when(s == _NS - 1)
            def _():
                pltpu.sync_copy(
                    acc_sh.at[pl.ds(_NS * rpt, tail)],
                    out_hbm.at[c, pl.ds(_NS * rpt, tail)],
                )

    return hist


@functools.cache
def _make_msg(np_, iters, d):
    rpt, tail = _row_split(np_)
    phases = []
    rem = iters
    while rem:
        t = min(40, rem)
        phases.append(t)
        rem -= t
    assert all(p >= 2 and p % 8 == 0 for p in phases)
    pmax = max(phases)

    @functools.partial(
        pl.kernel,
        out_type=jax.ShapeDtypeStruct((_NC, np_, d), jnp.float32),
        mesh=_sc_mesh(),
        scratch_types=[
            pltpu.VMEM((pmax, _K), jnp.int32),
            pltpu.VMEM((pmax, _K), jnp.int32),
            pltpu.VMEM((_K, d), jnp.float32),
            pltpu.VMEM((_K, d), jnp.float32),
            pltpu.VMEM_SHARED((np_, d), jnp.float32),
            pltpu.SemaphoreType.DMA,
            pltpu.SemaphoreType.DMA,
            pltpu.SemaphoreType.DMA,
            pltpu.SemaphoreType.DMA,
        ],
    )
    def msg(hs_hbm, src_hbm, dst_hbm, zeros_hbm, out_hbm,
            sidx_v, didx_v, rows0_v, rows1_v, acc_sh, g0, g1, s0, s1):
        c = lax.axis_index("c")
        s = lax.axis_index("s")
        wid = s * _NC + c
        pltpu.sync_copy(
            zeros_hbm.at[pl.ds(s * rpt, rpt)], acc_sh.at[pl.ds(s * rpt, rpt)]
        )
        if tail:
            @pl.when(s == _NS - 1)
            def _():
                pltpu.sync_copy(
                    zeros_hbm.at[pl.ds(_NS * rpt, tail)],
                    acc_sh.at[pl.ds(_NS * rpt, tail)],
                )
        plsc.subcore_barrier()

        def g_start(j, rows_v, sem):
            pltpu.async_copy(hs_hbm.at[sidx_v.at[j]], rows_v, sem)

        def g_wait(rows_v, sem):
            pltpu.make_async_copy(hs_hbm.at[sidx_v.at[0]], rows_v, sem).wait()

        def s_start(j, rows_v, sem):
            pltpu.async_copy(rows_v, acc_sh.at[didx_v.at[j]], sem, add=True)

        def s_wait(rows_v, sem):
            pltpu.make_async_copy(rows_v, acc_sh.at[didx_v.at[0]], sem).wait()

        base = 0
        for count in phases:
            pltpu.sync_copy(
                src_hbm.at[wid, pl.ds(base, count)], sidx_v.at[pl.ds(0, count)]
            )
            pltpu.sync_copy(
                dst_hbm.at[wid, pl.ds(base, count)], didx_v.at[pl.ds(0, count)]
            )
            g_start(0, rows0_v, g0)

            def body(jj, carry):
                j2 = jj * 2
                g_wait(rows0_v, g0)
                @pl.when(jj > 0)
                def _():
                    s_wait(rows1_v, s1)
                g_start(j2 + 1, rows1_v, g1)
                s_start(j2, rows0_v, s0)
                g_wait(rows1_v, g1)
                s_wait(rows0_v, s0)
                g_start(jnp.minimum(j2 + 2, count - 1), rows0_v, g0)
                s_start(j2 + 1, rows1_v, s1)
                return carry

            lax.fori_loop(0, count // 2, body, 0)
            g_wait(rows0_v, g0)
            s_wait(rows1_v, s1)
            base += count
        plsc.subcore_barrier()
        pltpu.sync_copy(
            acc_sh.at[pl.ds(s * rpt, rpt)], out_hbm.at[c, pl.ds(s * rpt, rpt)]
        )
        if tail:
            @pl.when(s == _NS - 1)
            def _():
                pltpu.sync_copy(
                    acc_sh.at[pl.ds(_NS * rpt, tail)],
                    out_hbm.at[c, pl.ds(_NS * rpt, tail)],
                )

    return msg


def _scale_matmul_body(x_ref, cnt_ref, w_ref, o_ref):
    cnt = cnt_ref[...]
    deg = cnt[0, :, 0:1] + cnt[1, :, 0:1] + 1.0
    dinv = lax.rsqrt(deg)
    o_ref[...] = jnp.dot(
        x_ref[...] * dinv, w_ref[...], preferred_element_type=jnp.float32
    )


def _scale_matmul(x, cnt, w):
    n, d_in = x.shape
    d_h = w.shape[1]
    bm = 1000
    grid = n // bm
    return pl.pallas_call(
        _scale_matmul_body,
        grid=(grid,),
        in_specs=[
            pl.BlockSpec((bm, d_in), lambda i: (i, 0)),
            pl.BlockSpec((_NC, bm, 128), lambda i: (0, i, 0)),
            pl.BlockSpec((d_in, d_h), lambda i: (0, 0)),
        ],
        out_specs=pl.BlockSpec((bm, d_h), lambda i: (i, 0)),
        out_shape=jax.ShapeDtypeStruct((n, d_h), jnp.float32),
    )(x, cnt, w)


def _make_finalize(n):
    def _finalize_body(s_ref, hs_ref, cnt_ref, b_ref, g_ref, be_ref, z_ref):
        cnt = cnt_ref[...]
        deg = cnt[0, :n, 0:1] + cnt[1, :n, 0:1] + 1.0
        dinv = lax.rsqrt(deg)
        ssum = s_ref[0, :n, :] + s_ref[1, :n, :]
        agg = dinv * (ssum + hs_ref[...]) + b_ref[...]
        inv_n = 1.0 / n
        mean = jnp.sum(agg, axis=0, keepdims=True) * inv_n
        cen = agg - mean
        var = jnp.sum(cen * cen, axis=0, keepdims=True) * inv_n
        z = cen * lax.rsqrt(var + 1e-5) * g_ref[...] + be_ref[...]
        z_ref[...] = jnp.maximum(z, 0.0)

    return _finalize_body


def _finalize(s, hs, cnt, b, g, be):
    n, d = hs.shape
    return pl.pallas_call(
        _make_finalize(n),
        out_shape=jax.ShapeDtypeStruct((n, d), jnp.float32),
    )(s, hs, cnt, b, g, be)


def _decoder_body(zi_ref, zj_ref, o_ref):
    o_ref[...] = lax.dot_general(
        zi_ref[...],
        zj_ref[...],
        (((1,), (1,)), ((), ())),
        preferred_element_type=jnp.float32,
    )


def _decoder(z):
    n, d = z.shape
    bm = 512
    grid = pl.cdiv(n, bm)
    return pl.pallas_call(
        _decoder_body,
        grid=(grid, grid),
        in_specs=[
            pl.BlockSpec((bm, d), lambda i, j: (i, 0)),
            pl.BlockSpec((bm, d), lambda i, j: (j, 0)),
        ],
        out_specs=pl.BlockSpec((bm, bm), lambda i, j: (i, j)),
        out_shape=jax.ShapeDtypeStruct((n, n), jnp.float32),
        compiler_params=pltpu.CompilerParams(
            dimension_semantics=("parallel", "parallel")
        ),
    )(z, z)


def kernel(x, edge_index, W, b, gamma, beta):
    n, _ = x.shape
    d_h = W.shape[1]
    e = edge_index.shape[1]
    dummy = 512
    np_ = (n + dummy + 7) // 8 * 8
    chunks = -(-e // (_NW * _K))
    iters = -(-chunks // 8) * 8
    e_pad = _NW * _K * iters
    ppt = (e_pad - e) // _NW
    ar = jnp.arange(_NW * ppt, dtype=jnp.int32).reshape(_NW, ppt)
    src = jnp.concatenate(
        [edge_index[0].reshape(_NW, -1), ar % dummy], axis=1
    ).reshape(_NW, iters, _K)
    dst = jnp.concatenate(
        [edge_index[1].reshape(_NW, -1), n + ar % dummy], axis=1
    ).reshape(_NW, iters, _K)
    dst_flat = dst.reshape(-1)
    zeros_d = jnp.zeros((np_, d_h), jnp.float32)
    cnt = _make_hist(np_, iters * _K)(dst_flat, zeros_d)
    hs = _scale_matmul(x, cnt, W)
    s = _make_msg(np_, iters, d_h)(hs, src, dst, zeros_d)
    z = _finalize(
        s, hs, cnt, b.reshape(1, d_h), gamma.reshape(1, d_h), beta.reshape(1, d_h)
    )
    return _decoder(z)


# speedup vs baseline: 1.6836x; 1.0199x over previous
import functools

import jax
import jax.numpy as jnp
from jax import lax
from jax.experimental import pallas as pl
from jax.experimental.pallas import tpu as pltpu
from jax.experimental.pallas import tpu_sc as plsc

_NC = 2
_NS = 16
_NW = _NC * _NS
_K = 80


def _sc_mesh():
    return plsc.VectorSubcoreMesh(
        core_axis_name="c", subcore_axis_name="s", num_cores=_NC, num_subcores=_NS
    )


def _row_split(np_):
    rpt = (np_ // _NS) // 8 * 8
    tail = np_ - _NS * rpt
    return rpt, tail


@functools.cache
def _make_hist(np_, epw):
    kh = 128
    iters = epw // kh
    assert iters % 2 == 0
    rpt, tail = _row_split(np_)

    @functools.partial(
        pl.kernel,
        out_type=jax.ShapeDtypeStruct((_NC, np_, 128), jnp.float32),
        mesh=_sc_mesh(),
        scratch_types=[
            pltpu.VMEM((kh,), jnp.int32),
            pltpu.VMEM((kh,), jnp.int32),
            pltpu.VMEM((kh, 128), jnp.float32),
            pltpu.VMEM_SHARED((np_, 128), jnp.float32),
            pltpu.SemaphoreType.DMA,
            pltpu.SemaphoreType.DMA,
        ],
    )
    def hist(dst_hbm, zeros_hbm, out_hbm, idxa_v, idxb_v, ones_v, acc_sh, la, lb):
        c = lax.axis_index("c")
        s = lax.axis_index("s")
        wid = s * _NC + c
        for r in range(kh):
            for q in range(8):
                ones_v[r, pl.ds(q * 16, 16)] = jnp.ones((16,), jnp.float32)
        pltpu.sync_copy(
            zeros_hbm.at[pl.ds(s * rpt, rpt)], acc_sh.at[pl.ds(s * rpt, rpt)]
        )
        if tail:
            @pl.when(s == _NS - 1)
            def _():
                pltpu.sync_copy(
                    zeros_hbm.at[pl.ds(_NS * rpt, tail)],
                    acc_sh.at[pl.ds(_NS * rpt, tail)],
                )
        plsc.subcore_barrier()

        def load(chunk, idx_v, sem):
            pltpu.async_copy(
                dst_hbm.at[pl.ds(wid * epw + chunk * kh, kh)], idx_v, sem
            )

        def load_wait(idx_v, sem):
            pltpu.make_async_copy(dst_hbm.at[pl.ds(0, kh)], idx_v, sem).wait()

        pltpu.sync_copy(dst_hbm.at[pl.ds(wid * epw, kh)], idxa_v)

        def body(j, carry):
            j2 = j * 2
            load(j2 + 1, idxb_v, lb)
            pltpu.sync_copy(ones_v, acc_sh.at[idxa_v], add=True)
            load_wait(idxb_v, lb)
            load(jnp.minimum(j2 + 2, iters - 1), idxa_v, la)
            pltpu.sync_copy(ones_v, acc_sh.at[idxb_v], add=True)
            load_wait(idxa_v, la)
            return carry

        lax.fori_loop(0, iters // 2, body, 0)
        plsc.subcore_barrier()
        pltpu.sync_copy(
            acc_sh.at[pl.ds(s * rpt, rpt)], out_hbm.at[c, pl.ds(s * rpt, rpt)]
        )
        if tail:
            @pl.when(s == _NS - 1)
            def _():
                pltpu.sync_copy(
                    acc_sh.at[pl.ds(_NS * rpt, tail)],
                    out_hbm.at[c, pl.ds(_NS * rpt, tail)],
                )

    return hist


@functools.cache
def _make_msg(np_, iters, d):
    rpt, tail = _row_split(np_)
    phases = []
    rem = iters
    while rem:
        t = min(40, rem)
        phases.append(t)
        rem -= t
    assert all(p >= 2 and p % 8 == 0 for p in phases)
    pmax = max(phases)

    @functools.partial(
        pl.kernel,
        out_type=jax.ShapeDtypeStruct((_NC, np_, d), jnp.float32),
        mesh=_sc_mesh(),
        scratch_types=[
            pltpu.VMEM((pmax, _K), jnp.int32),
            pltpu.VMEM((pmax, _K), jnp.int32),
            pltpu.VMEM((_K, d), jnp.float32),
            pltpu.VMEM((_K, d), jnp.float32),
            pltpu.VMEM_SHARED((np_, d), jnp.float32),
            pltpu.SemaphoreType.DMA,
            pltpu.SemaphoreType.DMA,
            pltpu.SemaphoreType.DMA,
            pltpu.SemaphoreType.DMA,
        ],
    )
    def msg(hs_hbm, src_hbm, dst_hbm, zeros_hbm, out_hbm,
            sidx_v, didx_v, rows0_v, rows1_v, acc_sh, g0, g1, s0, s1):
        c = lax.axis_index("c")
        s = lax.axis_index("s")
        wid = s * _NC + c
        pltpu.sync_copy(
            zeros_hbm.at[pl.ds(s * rpt, rpt)], acc_sh.at[pl.ds(s * rpt, rpt)]
        )
        if tail:
            @pl.when(s == _NS - 1)
            def _():
                pltpu.sync_copy(
                    zeros_hbm.at[pl.ds(_NS * rpt, tail)],
                    acc_sh.at[pl.ds(_NS * rpt, tail)],
                )
        plsc.subcore_barrier()

        def g_start(j, rows_v, sem):
            pltpu.async_copy(hs_hbm.at[sidx_v.at[j]], rows_v, sem)

        def g_wait(rows_v, sem):
            pltpu.make_async_copy(hs_hbm.at[sidx_v.at[0]], rows_v, sem).wait()

        def s_start(j, rows_v, sem):
            pltpu.async_copy(rows_v, acc_sh.at[didx_v.at[j]], sem, add=True)

        def s_wait(rows_v, sem):
            pltpu.make_async_copy(rows_v, acc_sh.at[didx_v.at[0]], sem).wait()

        base = 0
        for count in phases:
            pltpu.sync_copy(
                src_hbm.at[wid, pl.ds(base, count)], sidx_v.at[pl.ds(0, count)]
            )
            pltpu.sync_copy(
                dst_hbm.at[wid, pl.ds(base, count)], didx_v.at[pl.ds(0, count)]
            )
            g_start(0, rows0_v, g0)

            def body(jj, carry):
                j2 = jj * 2
                g_wait(rows0_v, g0)
                @pl.when(jj > 0)
                def _():
                    s_wait(rows1_v, s1)
                g_start(j2 + 1, rows1_v, g1)
                s_start(j2, rows0_v, s0)
                g_wait(rows1_v, g1)
                s_wait(rows0_v, s0)
                g_start(jnp.minimum(j2 + 2, count - 1), rows0_v, g0)
                s_start(j2 + 1, rows1_v, s1)
                return carry

            lax.fori_loop(0, count // 2, body, 0)
            g_wait(rows0_v, g0)
            s_wait(rows1_v, s1)
            base += count
        plsc.subcore_barrier()
        pltpu.sync_copy(
            acc_sh.at[pl.ds(s * rpt, rpt)], out_hbm.at[c, pl.ds(s * rpt, rpt)]
        )
        if tail:
            @pl.when(s == _NS - 1)
            def _():
                pltpu.sync_copy(
                    acc_sh.at[pl.ds(_NS * rpt, tail)],
                    out_hbm.at[c, pl.ds(_NS * rpt, tail)],
                )

    return msg


def _scale_matmul_body(x_ref, cnt_ref, w_ref, o_ref):
    cnt = cnt_ref[...]
    deg = cnt[0, :, 0:1] + cnt[1, :, 0:1] + 1.0
    dinv = lax.rsqrt(deg)
    o_ref[...] = jnp.dot(
        x_ref[...] * dinv, w_ref[...], preferred_element_type=jnp.float32
    )


def _scale_matmul(x, cnt, w):
    n, d_in = x.shape
    d_h = w.shape[1]
    bm = 1000
    grid = n // bm
    return pl.pallas_call(
        _scale_matmul_body,
        grid=(grid,),
        in_specs=[
            pl.BlockSpec((bm, d_in), lambda i: (i, 0)),
            pl.BlockSpec((_NC, bm, 128), lambda i: (0, i, 0)),
            pl.BlockSpec((d_in, d_h), lambda i: (0, 0)),
        ],
        out_specs=pl.BlockSpec((bm, d_h), lambda i: (i, 0)),
        out_shape=jax.ShapeDtypeStruct((n, d_h), jnp.float32),
    )(x, cnt, w)


def _make_finalize(n):
    def _finalize_body(s_ref, hs_ref, cnt_ref, b_ref, g_ref, be_ref, z_ref):
        cnt = cnt_ref[...]
        deg = cnt[0, :n, 0:1] + cnt[1, :n, 0:1] + 1.0
        dinv = lax.rsqrt(deg)
        ssum = s_ref[0, :n, :] + s_ref[1, :n, :]
        agg = dinv * (ssum + hs_ref[...]) + b_ref[...]
        inv_n = 1.0 / n
        mean = jnp.sum(agg, axis=0, keepdims=True) * inv_n
        cen = agg - mean
        var = jnp.sum(cen * cen, axis=0, keepdims=True) * inv_n
        z = cen * lax.rsqrt(var + 1e-5) * g_ref[...] + be_ref[...]
        z_ref[...] = jnp.maximum(z, 0.0)

    return _finalize_body


def _finalize(s, hs, cnt, b, g, be):
    n, d = hs.shape
    return pl.pallas_call(
        _make_finalize(n),
        out_shape=jax.ShapeDtypeStruct((n, d), jnp.float32),
    )(s, hs, cnt, b, g, be)


def _decoder_body(zi_ref, zj_ref, o_ref):
    o_ref[...] = lax.dot_general(
        zi_ref[...],
        zj_ref[...],
        (((1,), (1,)), ((), ())),
        preferred_element_type=jnp.float32,
    )


def _decoder(z):
    n, d = z.shape
    z = z.astype(jnp.bfloat16)
    bm = 512
    grid = pl.cdiv(n, bm)
    return pl.pallas_call(
        _decoder_body,
        grid=(grid, grid),
        in_specs=[
            pl.BlockSpec((bm, d), lambda i, j: (i, 0)),
            pl.BlockSpec((bm, d), lambda i, j: (j, 0)),
        ],
        out_specs=pl.BlockSpec((bm, bm), lambda i, j: (i, j)),
        out_shape=jax.ShapeDtypeStruct((n, n), jnp.float32),
        compiler_params=pltpu.CompilerParams(
            dimension_semantics=("parallel", "parallel")
        ),
    )(z, z)


def kernel(x, edge_index, W, b, gamma, beta):
    n, _ = x.shape
    d_h = W.shape[1]
    e = edge_index.shape[1]
    dummy = 512
    np_ = (n + dummy + 7) // 8 * 8
    chunks = -(-e // (_NW * _K))
    iters = -(-chunks // 8) * 8
    e_pad = _NW * _K * iters
    ppt = (e_pad - e) // _NW
    ar = jnp.arange(_NW * ppt, dtype=jnp.int32).reshape(_NW, ppt)
    src = jnp.concatenate(
        [edge_index[0].reshape(_NW, -1), ar % dummy], axis=1
    ).reshape(_NW, iters, _K)
    dst = jnp.concatenate(
        [edge_index[1].reshape(_NW, -1), n + ar % dummy], axis=1
    ).reshape(_NW, iters, _K)
    dst_flat = dst.reshape(-1)
    zeros_d = jnp.zeros((np_, d_h), jnp.float32)
    cnt = _make_hist(np_, iters * _K)(dst_flat, zeros_d)
    hs = _scale_matmul(x, cnt, W)
    s = _make_msg(np_, iters, d_h)(hs, src, dst, zeros_d)
    z = _finalize(
        s, hs, cnt, b.reshape(1, d_h), gamma.reshape(1, d_h), beta.reshape(1, d_h)
    )
    return _decoder(z)


# speedup vs baseline: 1.7322x; 1.0289x over previous
import functools

import jax
import jax.numpy as jnp
from jax import lax
from jax.experimental import pallas as pl
from jax.experimental.pallas import tpu as pltpu
from jax.experimental.pallas import tpu_sc as plsc

_NC = 2
_NS = 16
_NW = _NC * _NS
_K = 128


def _sc_mesh():
    return plsc.VectorSubcoreMesh(
        core_axis_name="c", subcore_axis_name="s", num_cores=_NC, num_subcores=_NS
    )


def _row_split(np_):
    rpt = (np_ // _NS) // 8 * 8
    tail = np_ - _NS * rpt
    return rpt, tail


@functools.cache
def _make_hist(np_, epw):
    kh = 128
    iters = epw // kh
    assert iters % 2 == 0
    rpt, tail = _row_split(np_)

    @functools.partial(
        pl.kernel,
        out_type=jax.ShapeDtypeStruct((_NC, np_, 128), jnp.float32),
        mesh=_sc_mesh(),
        scratch_types=[
            pltpu.VMEM((kh,), jnp.int32),
            pltpu.VMEM((kh,), jnp.int32),
            pltpu.VMEM((kh, 128), jnp.float32),
            pltpu.VMEM_SHARED((np_, 128), jnp.float32),
            pltpu.SemaphoreType.DMA,
            pltpu.SemaphoreType.DMA,
        ],
    )
    def hist(dst_hbm, zeros_hbm, out_hbm, idxa_v, idxb_v, ones_v, acc_sh, la, lb):
        c = lax.axis_index("c")
        s = lax.axis_index("s")
        wid = s * _NC + c
        for r in range(kh):
            for q in range(8):
                ones_v[r, pl.ds(q * 16, 16)] = jnp.ones((16,), jnp.float32)
        pltpu.sync_copy(
            zeros_hbm.at[pl.ds(s * rpt, rpt)], acc_sh.at[pl.ds(s * rpt, rpt)]
        )
        if tail:
            @pl.when(s == _NS - 1)
            def _():
                pltpu.sync_copy(
                    zeros_hbm.at[pl.ds(_NS * rpt, tail)],
                    acc_sh.at[pl.ds(_NS * rpt, tail)],
                )
        plsc.subcore_barrier()

        def load(chunk, idx_v, sem):
            pltpu.async_copy(
                dst_hbm.at[pl.ds(wid * epw + chunk * kh, kh)], idx_v, sem
            )

        def load_wait(idx_v, sem):
            pltpu.make_async_copy(dst_hbm.at[pl.ds(0, kh)], idx_v, sem).wait()

        pltpu.sync_copy(dst_hbm.at[pl.ds(wid * epw, kh)], idxa_v)

        def body(j, carry):
            j2 = j * 2
            load(j2 + 1, idxb_v, lb)
            pltpu.sync_copy(ones_v, acc_sh.at[idxa_v], add=True)
            load_wait(idxb_v, lb)
            load(jnp.minimum(j2 + 2, iters - 1), idxa_v, la)
            pltpu.sync_copy(ones_v, acc_sh.at[idxb_v], add=True)
            load_wait(idxa_v, la)
            return carry

        lax.fori_loop(0, iters // 2, body, 0)
        plsc.subcore_barrier()
        pltpu.sync_copy(
            acc_sh.at[pl.ds(s * rpt, rpt)], out_hbm.at[c, pl.ds(s * rpt, rpt)]
        )
        if tail:
            @pl.when(s == _NS - 1)
            def _():
                pltpu.sync_copy(
                    acc_sh.at[pl.ds(_NS * rpt, tail)],
                    out_hbm.at[c, pl.ds(_NS * rpt, tail)],
                )

    return hist


@functools.cache
def _make_msg(np_, iters, d):
    rpt, tail = _row_split(np_)
    phases = []
    rem = iters
    while rem:
        t = min(16, rem)
        phases.append(t)
        rem -= t
    assert all(p >= 2 and p % 8 == 0 for p in phases)
    pmax = max(phases)

    @functools.partial(
        pl.kernel,
        out_type=jax.ShapeDtypeStruct((_NC, np_, d), jnp.float32),
        mesh=_sc_mesh(),
        scratch_types=[
            pltpu.VMEM((pmax, _K), jnp.int32),
            pltpu.VMEM((pmax, _K), jnp.int32),
            pltpu.VMEM((_K, d), jnp.float32),
            pltpu.VMEM((_K, d), jnp.float32),
            pltpu.VMEM_SHARED((np_, d), jnp.float32),
            pltpu.SemaphoreType.DMA,
            pltpu.SemaphoreType.DMA,
            pltpu.SemaphoreType.DMA,
            pltpu.SemaphoreType.DMA,
        ],
    )
    def msg(hs_hbm, src_hbm, dst_hbm, zeros_hbm, out_hbm,
            sidx_v, didx_v, rows0_v, rows1_v, acc_sh, g0, g1, s0, s1):
        c = lax.axis_index("c")
        s = lax.axis_index("s")
        wid = s * _NC + c
        pltpu.sync_copy(
            zeros_hbm.at[pl.ds(s * rpt, rpt)], acc_sh.at[pl.ds(s * rpt, rpt)]
        )
        if tail:
            @pl.when(s == _NS - 1)
            def _():
                pltpu.sync_copy(
                    zeros_hbm.at[pl.ds(_NS * rpt, tail)],
                    acc_sh.at[pl.ds(_NS * rpt, tail)],
                )
        plsc.subcore_barrier()

        def g_start(j, rows_v, sem):
            pltpu.async_copy(hs_hbm.at[sidx_v.at[j]], rows_v, sem)

        def g_wait(rows_v, sem):
            pltpu.make_async_copy(hs_hbm.at[sidx_v.at[0]], rows_v, sem).wait()

        def s_start(j, rows_v, sem):
            pltpu.async_copy(rows_v, acc_sh.at[didx_v.at[j]], sem, add=True)

        def s_wait(rows_v, sem):
            pltpu.make_async_copy(rows_v, acc_sh.at[didx_v.at[0]], sem).wait()

        base = 0
        for count in phases:
            pltpu.sync_copy(
                src_hbm.at[wid, pl.ds(base, count)], sidx_v.at[pl.ds(0, count)]
            )
            pltpu.sync_copy(
                dst_hbm.at[wid, pl.ds(base, count)], didx_v.at[pl.ds(0, count)]
            )
            g_start(0, rows0_v, g0)

            def body(jj, carry):
                j2 = jj * 2
                g_wait(rows0_v, g0)
                @pl.when(jj > 0)
                def _():
                    s_wait(rows1_v, s1)
                g_start(j2 + 1, rows1_v, g1)
                s_start(j2, rows0_v, s0)
                g_wait(rows1_v, g1)
                s_wait(rows0_v, s0)
                g_start(jnp.minimum(j2 + 2, count - 1), rows0_v, g0)
                s_start(j2 + 1, rows1_v, s1)
                return carry

            lax.fori_loop(0, count // 2, body, 0)
            g_wait(rows0_v, g0)
            s_wait(rows1_v, s1)
            base += count
        plsc.subcore_barrier()
        pltpu.sync_copy(
            acc_sh.at[pl.ds(s * rpt, rpt)], out_hbm.at[c, pl.ds(s * rpt, rpt)]
        )
        if tail:
            @pl.when(s == _NS - 1)
            def _():
                pltpu.sync_copy(
                    acc_sh.at[pl.ds(_NS * rpt, tail)],
                    out_hbm.at[c, pl.ds(_NS * rpt, tail)],
                )

    return msg


def _scale_matmul_body(x_ref, cnt_ref, w_ref, o_ref):
    cnt = cnt_ref[...]
    deg = cnt[0, :, 0:1] + cnt[1, :, 0:1] + 1.0
    dinv = lax.rsqrt(deg)
    o_ref[...] = jnp.dot(
        x_ref[...] * dinv, w_ref[...], preferred_element_type=jnp.float32
    )


def _scale_matmul(x, cnt, w):
    n, d_in = x.shape
    d_h = w.shape[1]
    bm = 1000
    grid = n // bm
    return pl.pallas_call(
        _scale_matmul_body,
        grid=(grid,),
        in_specs=[
            pl.BlockSpec((bm, d_in), lambda i: (i, 0)),
            pl.BlockSpec((_NC, bm, 128), lambda i: (0, i, 0)),
            pl.BlockSpec((d_in, d_h), lambda i: (0, 0)),
        ],
        out_specs=pl.BlockSpec((bm, d_h), lambda i: (i, 0)),
        out_shape=jax.ShapeDtypeStruct((n, d_h), jnp.float32),
    )(x, cnt, w)


def _make_finalize(n):
    def _finalize_body(s_ref, hs_ref, cnt_ref, b_ref, g_ref, be_ref, z_ref):
        cnt = cnt_ref[...]
        deg = cnt[0, :n, 0:1] + cnt[1, :n, 0:1] + 1.0
        dinv = lax.rsqrt(deg)
        ssum = s_ref[0, :n, :] + s_ref[1, :n, :]
        agg = dinv * (ssum + hs_ref[...]) + b_ref[...]
        inv_n = 1.0 / n
        mean = jnp.sum(agg, axis=0, keepdims=True) * inv_n
        cen = agg - mean
        var = jnp.sum(cen * cen, axis=0, keepdims=True) * inv_n
        z = cen * lax.rsqrt(var + 1e-5) * g_ref[...] + be_ref[...]
        z_ref[...] = jnp.maximum(z, 0.0)

    return _finalize_body


def _finalize(s, hs, cnt, b, g, be):
    n, d = hs.shape
    return pl.pallas_call(
        _make_finalize(n),
        out_shape=jax.ShapeDtypeStruct((n, d), jnp.float32),
    )(s, hs, cnt, b, g, be)


def _decoder_body(zi_ref, zj_ref, o_ref):
    o_ref[...] = lax.dot_general(
        zi_ref[...],
        zj_ref[...],
        (((1,), (1,)), ((), ())),
        preferred_element_type=jnp.float32,
    )


def _decoder(z):
    n, d = z.shape
    z = z.astype(jnp.bfloat16)
    bm = 512
    grid = pl.cdiv(n, bm)
    return pl.pallas_call(
        _decoder_body,
        grid=(grid, grid),
        in_specs=[
            pl.BlockSpec((bm, d), lambda i, j: (i, 0)),
            pl.BlockSpec((bm, d), lambda i, j: (j, 0)),
        ],
        out_specs=pl.BlockSpec((bm, bm), lambda i, j: (i, j)),
        out_shape=jax.ShapeDtypeStruct((n, n), jnp.float32),
        compiler_params=pltpu.CompilerParams(
            dimension_semantics=("parallel", "parallel")
        ),
    )(z, z)


def kernel(x, edge_index, W, b, gamma, beta):
    n, _ = x.shape
    d_h = W.shape[1]
    e = edge_index.shape[1]
    dummy = 64
    np_ = (n + dummy + 7) // 8 * 8
    chunks = -(-e // (_NW * _K))
    iters = -(-chunks // 8) * 8
    e_pad = _NW * _K * iters
    ppt = (e_pad - e) // _NW
    ar = jnp.arange(_NW * ppt, dtype=jnp.int32).reshape(_NW, ppt)
    src = jnp.concatenate(
        [edge_index[0].reshape(_NW, -1), ar % dummy], axis=1
    ).reshape(_NW, iters, _K)
    dst = jnp.concatenate(
        [edge_index[1].reshape(_NW, -1), n + ar % dummy], axis=1
    ).reshape(_NW, iters, _K)
    dst_flat = dst.reshape(-1)
    zeros_d = jnp.zeros((np_, d_h), jnp.float32)
    cnt = _make_hist(np_, iters * _K)(dst_flat, zeros_d)
    hs = _scale_matmul(x, cnt, W)
    s = _make_msg(np_, iters, d_h)(hs, src, dst, zeros_d)
    z = _finalize(
        s, hs, cnt, b.reshape(1, d_h), gamma.reshape(1, d_h), beta.reshape(1, d_h)
    )
    return _decoder(z)


# speedup vs baseline: 2.1472x; 1.2395x over previous
import functools

import jax
import jax.numpy as jnp
from jax import lax
from jax.experimental import pallas as pl
from jax.experimental.pallas import tpu as pltpu
from jax.experimental.pallas import tpu_sc as plsc

_NC = 2
_NS = 16
_NW = _NC * _NS
_K = 128


def _sc_mesh():
    return plsc.VectorSubcoreMesh(
        core_axis_name="c", subcore_axis_name="s", num_cores=_NC, num_subcores=_NS
    )


def _row_split(np_):
    rpt = (np_ // _NS) // 8 * 8
    tail = np_ - _NS * rpt
    return rpt, tail


@functools.cache
def _make_hist(np_, epw):
    kh = 128
    iters = epw // kh
    assert iters % 2 == 0
    rpt, tail = _row_split(np_)

    @functools.partial(
        pl.kernel,
        out_type=jax.ShapeDtypeStruct((_NC, np_, 128), jnp.float32),
        mesh=_sc_mesh(),
        scratch_types=[
            pltpu.VMEM((kh,), jnp.int32),
            pltpu.VMEM((kh,), jnp.int32),
            pltpu.VMEM((kh, 128), jnp.float32),
            pltpu.VMEM_SHARED((np_, 128), jnp.float32),
            pltpu.SemaphoreType.DMA,
            pltpu.SemaphoreType.DMA,
        ],
    )
    def hist(dst_hbm, zeros_hbm, out_hbm, idxa_v, idxb_v, ones_v, acc_sh, la, lb):
        c = lax.axis_index("c")
        s = lax.axis_index("s")
        wid = s * _NC + c
        for r in range(kh):
            for q in range(8):
                ones_v[r, pl.ds(q * 16, 16)] = jnp.ones((16,), jnp.float32)
        pltpu.sync_copy(
            zeros_hbm.at[pl.ds(s * rpt, rpt)], acc_sh.at[pl.ds(s * rpt, rpt)]
        )
        if tail:
            @pl.when(s == _NS - 1)
            def _():
                pltpu.sync_copy(
                    zeros_hbm.at[pl.ds(_NS * rpt, tail)],
                    acc_sh.at[pl.ds(_NS * rpt, tail)],
                )
        plsc.subcore_barrier()

        def load(chunk, idx_v, sem):
            pltpu.async_copy(
                dst_hbm.at[pl.ds(wid * epw + chunk * kh, kh)], idx_v, sem
            )

        def load_wait(idx_v, sem):
            pltpu.make_async_copy(dst_hbm.at[pl.ds(0, kh)], idx_v, sem).wait()

        pltpu.sync_copy(dst_hbm.at[pl.ds(wid * epw, kh)], idxa_v)

        def body(j, carry):
            j2 = j * 2
            load(j2 + 1, idxb_v, lb)
            pltpu.sync_copy(ones_v, acc_sh.at[idxa_v], add=True)
            load_wait(idxb_v, lb)
            load(jnp.minimum(j2 + 2, iters - 1), idxa_v, la)
            pltpu.sync_copy(ones_v, acc_sh.at[idxb_v], add=True)
            load_wait(idxa_v, la)
            return carry

        lax.fori_loop(0, iters // 2, body, 0)
        plsc.subcore_barrier()
        pltpu.sync_copy(
            acc_sh.at[pl.ds(s * rpt, rpt)], out_hbm.at[c, pl.ds(s * rpt, rpt)]
        )
        if tail:
            @pl.when(s == _NS - 1)
            def _():
                pltpu.sync_copy(
                    acc_sh.at[pl.ds(_NS * rpt, tail)],
                    out_hbm.at[c, pl.ds(_NS * rpt, tail)],
                )

    return hist


@functools.cache
def _make_msg(np_, iters, d):
    rpt, tail = _row_split(np_)
    phases = []
    rem = iters
    while rem:
        t = min(16, rem)
        phases.append(t)
        rem -= t
    assert all(p >= 2 and p % 8 == 0 for p in phases)
    pmax = max(phases)

    @functools.partial(
        pl.kernel,
        out_type=jax.ShapeDtypeStruct((_NC, np_, d), jnp.float32),
        mesh=_sc_mesh(),
        scratch_types=[
            pltpu.VMEM((pmax, _K), jnp.int32),
            pltpu.VMEM((pmax, _K), jnp.int32),
            pltpu.VMEM((_K, d), jnp.float32),
            pltpu.VMEM((_K, d), jnp.float32),
            pltpu.VMEM_SHARED((np_, d), jnp.float32),
            pltpu.SemaphoreType.DMA,
            pltpu.SemaphoreType.DMA,
            pltpu.SemaphoreType.DMA,
            pltpu.SemaphoreType.DMA,
        ],
    )
    def msg(hs_hbm, src_hbm, dst_hbm, zeros_hbm, out_hbm,
            sidx_v, didx_v, rows0_v, rows1_v, acc_sh, g0, g1, s0, s1):
        c = lax.axis_index("c")
        s = lax.axis_index("s")
        wid = s * _NC + c
        pltpu.sync_copy(
            zeros_hbm.at[pl.ds(s * rpt, rpt)], acc_sh.at[pl.ds(s * rpt, rpt)]
        )
        if tail:
            @pl.when(s == _NS - 1)
            def _():
                pltpu.sync_copy(
                    zeros_hbm.at[pl.ds(_NS * rpt, tail)],
                    acc_sh.at[pl.ds(_NS * rpt, tail)],
                )
        plsc.subcore_barrier()

        def g_start(j, rows_v, sem):
            pltpu.async_copy(hs_hbm.at[sidx_v.at[j]], rows_v, sem)

        def g_wait(rows_v, sem):
            pltpu.make_async_copy(hs_hbm.at[sidx_v.at[0]], rows_v, sem).wait()

        def s_start(j, rows_v, sem):
            pltpu.async_copy(rows_v, acc_sh.at[didx_v.at[j]], sem, add=True)

        def s_wait(rows_v, sem):
            pltpu.make_async_copy(rows_v, acc_sh.at[didx_v.at[0]], sem).wait()

        base = 0
        for count in phases:
            pltpu.sync_copy(
                src_hbm.at[wid, pl.ds(base, count)], sidx_v.at[pl.ds(0, count)]
            )
            pltpu.sync_copy(
                dst_hbm.at[wid, pl.ds(base, count)], didx_v.at[pl.ds(0, count)]
            )
            g_start(0, rows0_v, g0)

            def body(jj, carry):
                j2 = jj * 2
                g_wait(rows0_v, g0)
                @pl.when(jj > 0)
                def _():
                    s_wait(rows1_v, s1)
                g_start(j2 + 1, rows1_v, g1)
                s_start(j2, rows0_v, s0)
                g_wait(rows1_v, g1)
                s_wait(rows0_v, s0)
                g_start(jnp.minimum(j2 + 2, count - 1), rows0_v, g0)
                s_start(j2 + 1, rows1_v, s1)
                return carry

            lax.fori_loop(0, count // 2, body, 0)
            g_wait(rows0_v, g0)
            s_wait(rows1_v, s1)
            base += count
        plsc.subcore_barrier()
        pltpu.sync_copy(
            acc_sh.at[pl.ds(s * rpt, rpt)], out_hbm.at[c, pl.ds(s * rpt, rpt)]
        )
        if tail:
            @pl.when(s == _NS - 1)
            def _():
                pltpu.sync_copy(
                    acc_sh.at[pl.ds(_NS * rpt, tail)],
                    out_hbm.at[c, pl.ds(_NS * rpt, tail)],
                )

    return msg


def _scale_matmul_body(x_ref, cnt_ref, w_ref, o_ref):
    cnt = cnt_ref[...]
    deg = cnt[0, :, 0:1] + cnt[1, :, 0:1] + 1.0
    dinv = lax.rsqrt(deg)
    o_ref[...] = jnp.dot(
        x_ref[...] * dinv, w_ref[...], preferred_element_type=jnp.float32
    )


def _scale_matmul(x, cnt, w):
    n, d_in = x.shape
    d_h = w.shape[1]
    bm = 1000
    grid = n // bm
    return pl.pallas_call(
        _scale_matmul_body,
        grid=(grid,),
        in_specs=[
            pl.BlockSpec((bm, d_in), lambda i: (i, 0)),
            pl.BlockSpec((_NC, bm, 128), lambda i: (0, i, 0)),
            pl.BlockSpec((d_in, d_h), lambda i: (0, 0)),
        ],
        out_specs=pl.BlockSpec((bm, d_h), lambda i: (i, 0)),
        out_shape=jax.ShapeDtypeStruct((n, d_h), jnp.float32),
    )(x, cnt, w)


def _make_finalize(n):
    def _finalize_body(s_ref, hs_ref, cnt_ref, b_ref, g_ref, be_ref, z_ref):
        cnt = cnt_ref[...]
        deg = cnt[0, :n, 0:1] + cnt[1, :n, 0:1] + 1.0
        dinv = lax.rsqrt(deg)
        ssum = s_ref[0, :n, :] + s_ref[1, :n, :]
        agg = dinv * (ssum + hs_ref[...]) + b_ref[...]
        inv_n = 1.0 / n
        mean = jnp.sum(agg, axis=0, keepdims=True) * inv_n
        cen = agg - mean
        var = jnp.sum(cen * cen, axis=0, keepdims=True) * inv_n
        z = cen * lax.rsqrt(var + 1e-5) * g_ref[...] + be_ref[...]
        z_ref[...] = jnp.maximum(z, 0.0)

    return _finalize_body


def _finalize(s, hs, cnt, b, g, be):
    n, d = hs.shape
    return pl.pallas_call(
        _make_finalize(n),
        out_shape=jax.ShapeDtypeStruct((n, d), jnp.float32),
    )(s, hs, cnt, b, g, be)


def _decoder_body(zi_ref, zj_ref, o_ref):
    o_ref[...] = lax.dot_general(
        zi_ref[...],
        zj_ref[...],
        (((1,), (1,)), ((), ())),
        preferred_element_type=jnp.float32,
    )


def _decoder(z):
    n, d = z.shape
    z = z.astype(jnp.bfloat16)
    bi, bj = 1024, 512
    return pl.pallas_call(
        _decoder_body,
        grid=(pl.cdiv(n, bi), pl.cdiv(n, bj)),
        in_specs=[
            pl.BlockSpec((bi, d), lambda i, j: (i, 0)),
            pl.BlockSpec((bj, d), lambda i, j: (j, 0)),
        ],
        out_specs=pl.BlockSpec((bi, bj), lambda i, j: (i, j)),
        out_shape=jax.ShapeDtypeStruct((n, n), jnp.float32),
        compiler_params=pltpu.CompilerParams(
            dimension_semantics=("parallel", "parallel")
        ),
    )(z, z)


def kernel(x, edge_index, W, b, gamma, beta):
    n, _ = x.shape
    d_h = W.shape[1]
    e = edge_index.shape[1]
    dummy = 64
    np_ = (n + dummy + 7) // 8 * 8
    chunks = -(-e // (_NW * _K))
    iters = -(-chunks // 8) * 8
    e_pad = _NW * _K * iters
    ppt = (e_pad - e) // _NW
    ar = jnp.arange(_NW * ppt, dtype=jnp.int32).reshape(_NW, ppt)
    src = jnp.concatenate(
        [edge_index[0].reshape(_NW, -1), ar % dummy], axis=1
    ).reshape(_NW, iters, _K)
    dst = jnp.concatenate(
        [edge_index[1].reshape(_NW, -1), n + ar % dummy], axis=1
    ).reshape(_NW, iters, _K)
    dst_flat = dst.reshape(-1)
    zeros_d = jnp.zeros((np_, d_h), jnp.float32)
    cnt = _make_hist(np_, iters * _K)(dst_flat, zeros_d)
    hs = _scale_matmul(x, cnt, W)
    s = _make_msg(np_, iters, d_h)(hs, src, dst, zeros_d)
    z = _finalize(
        s, hs, cnt, b.reshape(1, d_h), gamma.reshape(1, d_h), beta.reshape(1, d_h)
    )
    return _decoder(z)
